# baseline (speedup 1.0000x reference)
import dataclasses

import jax
import jax.numpy as jnp
from jax import lax
from jax.experimental import pallas as pl
from jax.experimental.pallas import tpu as pltpu
from jax.experimental.pallas import tpu_sc as plsc

_NUM_CORES = 2
_NUM_SUBCORES = 16
_NUM_WORKERS = _NUM_CORES * _NUM_SUBCORES
_CHUNK = 128
_LANES = 16


def kernel(inputs, lookup_table):
    batch, seq = inputs.shape
    vocab, dim = lookup_table.shape
    n = batch * seq
    rows_per_tile = 128 // dim
    n_chunks = n // _CHUNK
    chunks_per_worker = n_chunks // _NUM_WORKERS
    chunks_per_col = batch // _CHUNK
    assert n % (_NUM_WORKERS * _CHUNK) == 0 and batch % _CHUNK == 0

    mesh = plsc.VectorSubcoreMesh(core_axis_name="c", subcore_axis_name="s")
    idx = inputs.T.reshape(n).astype(jnp.int32)
    table4 = lookup_table.reshape(vocab // rows_per_tile, 128)

    cp = pltpu.CompilerParams()
    if "needs_layout_passes" in pltpu.CompilerParams.__dataclass_fields__:
        cp = dataclasses.replace(cp, needs_layout_passes=False)

    @pl.kernel(
        out_type=jax.ShapeDtypeStruct((seq * dim, batch), jnp.float32),
        mesh=mesh,
        compiler_params=cp,
        scratch_types=[
            pltpu.VMEM((_CHUNK,), jnp.int32),
            pltpu.VMEM((_CHUNK,), jnp.int32),
            pltpu.VMEM((_CHUNK, 128), jnp.float32),
            pltpu.VMEM((dim, _CHUNK), jnp.float32),
            pltpu.SemaphoreType.DMA,
        ],
    )
    def gather_kernel(table_hbm, idx_hbm, out_hbm, idx_v, r_v, rows_v, trans_v, sem):
        wid = lax.axis_index("s") * _NUM_CORES + lax.axis_index("c")
        iota = lax.iota(jnp.int32, _LANES)

        @pl.loop(0, chunks_per_worker)
        def _(lc):
            ch = wid * chunks_per_worker + lc
            t = ch // chunks_per_col
            b0 = (ch % chunks_per_col) * _CHUNK
            base = pl.multiple_of(ch * _CHUNK, _CHUNK)
            pltpu.sync_copy(idx_hbm.at[pl.ds(base, _CHUNK)], idx_v)
            for g in range(_CHUNK // _LANES):
                v = idx_v[pl.ds(g * _LANES, _LANES)]
                r_v[pl.ds(g * _LANES, _LANES)] = lax.shift_right_logical(v, 2)
            pltpu.async_copy(table_hbm.at[r_v], rows_v, sem).wait()
            for g in range(_CHUNK // _LANES):
                v = idx_v[pl.ds(g * _LANES, _LANES)]
                c16 = (v & 3) * dim
                row16 = g * _LANES + iota

                @pl.loop(0, dim)
                def _(e):
                    val = plsc.load_gather(rows_v, [row16, c16 + e])
                    trans_v[e, pl.ds(g * _LANES, _LANES)] = val

            r0 = pl.multiple_of(t * dim, 8)
            pltpu.sync_copy(
                trans_v, out_hbm.at[pl.ds(r0, dim), pl.ds(b0, _CHUNK)]
            )

    out = gather_kernel(table4, idx)
    return out.reshape(seq, dim, batch).transpose(2, 0, 1)


# speedup vs baseline: 1.2346x; 1.2346x over previous
import dataclasses

import jax
import jax.numpy as jnp
from jax import lax
from jax.experimental import pallas as pl
from jax.experimental.pallas import tpu as pltpu
from jax.experimental.pallas import tpu_sc as plsc

_NUM_CORES = 2
_NUM_SUBCORES = 16
_NUM_WORKERS = _NUM_CORES * _NUM_SUBCORES
_CHUNK = 128
_LANES = 16


def kernel(inputs, lookup_table):
    batch, seq = inputs.shape
    vocab, dim = lookup_table.shape
    n = batch * seq
    rows_per_tile = 128 // dim
    n_chunks = n // _CHUNK
    chunks_per_worker = n_chunks // _NUM_WORKERS
    chunks_per_col = batch // _CHUNK
    assert n % (_NUM_WORKERS * _CHUNK) == 0 and batch % _CHUNK == 0
    assert chunks_per_worker % 2 == 0

    mesh = plsc.VectorSubcoreMesh(core_axis_name="c", subcore_axis_name="s")
    idx = inputs.T.reshape(n).astype(jnp.int32)
    table4 = lookup_table.reshape(vocab // rows_per_tile, 128)

    cp = pltpu.CompilerParams()
    if "needs_layout_passes" in pltpu.CompilerParams.__dataclass_fields__:
        cp = dataclasses.replace(cp, needs_layout_passes=False)

    @pl.kernel(
        out_type=jax.ShapeDtypeStruct((seq * dim, batch), jnp.float32),
        mesh=mesh,
        compiler_params=cp,
        scratch_types=[
            pltpu.VMEM((_CHUNK,), jnp.int32),
            pltpu.VMEM((_CHUNK,), jnp.int32),
            pltpu.VMEM((_CHUNK,), jnp.int32),
            pltpu.VMEM((_CHUNK,), jnp.int32),
            pltpu.VMEM((_CHUNK, 128), jnp.float32),
            pltpu.VMEM((_CHUNK, 128), jnp.float32),
            pltpu.VMEM((dim, _CHUNK), jnp.float32),
            pltpu.VMEM((dim, _CHUNK), jnp.float32),
            pltpu.SemaphoreType.DMA,
            pltpu.SemaphoreType.DMA,
            pltpu.SemaphoreType.DMA,
            pltpu.SemaphoreType.DMA,
        ],
    )
    def gather_kernel(
        table_hbm,
        idx_hbm,
        out_hbm,
        idx_a,
        idx_b,
        r_a,
        r_b,
        rows_a,
        rows_b,
        trans_a,
        trans_b,
        sem_ga,
        sem_gb,
        sem_oa,
        sem_ob,
    ):
        wid = lax.axis_index("s") * _NUM_CORES + lax.axis_index("c")
        first = wid * chunks_per_worker
        iota = lax.iota(jnp.int32, _LANES)
        groups = _CHUNK // _LANES

        def fetch(ch, idx_v, r_v, rows_v, sem_g):
            base = pl.multiple_of(ch * _CHUNK, _CHUNK)
            pltpu.sync_copy(idx_hbm.at[pl.ds(base, _CHUNK)], idx_v)
            for g in range(groups):
                v = idx_v[pl.ds(g * _LANES, _LANES)]
                r_v[pl.ds(g * _LANES, _LANES)] = lax.shift_right_logical(v, 2)
            return pltpu.async_copy(table_hbm.at[r_v], rows_v, sem_g)

        def select_store(ch, idx_v, rows_v, trans_v, sem_o):
            for g in range(groups):
                v = idx_v[pl.ds(g * _LANES, _LANES)]
                col = (v & (rows_per_tile - 1)) * dim
                row16 = g * _LANES + iota
                for e in range(dim):
                    val = plsc.load_gather(rows_v, [row16, col])
                    trans_v[e, pl.ds(g * _LANES, _LANES)] = val
                    if e + 1 < dim:
                        col = col + 1
            t = ch // chunks_per_col
            b0 = (ch % chunks_per_col) * _CHUNK
            r0 = pl.multiple_of(t * dim, 8)
            return pltpu.async_copy(
                trans_v, out_hbm.at[pl.ds(r0, dim), pl.ds(b0, _CHUNK)], sem_o
            )

        fetch(first, idx_a, r_a, rows_a, sem_ga)

        @pl.loop(0, chunks_per_worker // 2)
        def _(k):
            cha = first + 2 * k
            chb = cha + 1
            fetch(chb, idx_b, r_b, rows_b, sem_gb)
            pltpu.make_async_copy(table_hbm.at[pl.ds(0, _CHUNK)], rows_a, sem_ga).wait()

            @pl.when(k > 0)
            def _():
                pltpu.make_async_copy(
                    trans_a, out_hbm.at[pl.ds(0, dim), pl.ds(0, _CHUNK)], sem_oa
                ).wait()

            select_store(cha, idx_a, rows_a, trans_a, sem_oa)

            @pl.when(k + 1 < chunks_per_worker // 2)
            def _():
                fetch(cha + 2, idx_a, r_a, rows_a, sem_ga)

            pltpu.make_async_copy(table_hbm.at[pl.ds(0, _CHUNK)], rows_b, sem_gb).wait()

            @pl.when(k > 0)
            def _():
                pltpu.make_async_copy(
                    trans_b, out_hbm.at[pl.ds(0, dim), pl.ds(0, _CHUNK)], sem_ob
                ).wait()

            select_store(chb, idx_b, rows_b, trans_b, sem_ob)

        pltpu.make_async_copy(
            trans_a, out_hbm.at[pl.ds(0, dim), pl.ds(0, _CHUNK)], sem_oa
        ).wait()
        pltpu.make_async_copy(
            trans_b, out_hbm.at[pl.ds(0, dim), pl.ds(0, _CHUNK)], sem_ob
        ).wait()

    out = gather_kernel(table4, idx)
    return out.reshape(seq, dim, batch).transpose(2, 0, 1)


# speedup vs baseline: 1.2996x; 1.0527x over previous
import dataclasses

import jax
import jax.numpy as jnp
from jax import lax
from jax.experimental import pallas as pl
from jax.experimental.pallas import tpu as pltpu
from jax.experimental.pallas import tpu_sc as plsc

_NUM_CORES = 2
_NUM_SUBCORES = 16
_NUM_WORKERS = _NUM_CORES * _NUM_SUBCORES
_CHUNK = 128
_LANES = 16


def kernel(inputs, lookup_table):
    batch, seq = inputs.shape
    vocab, dim = lookup_table.shape
    n = batch * seq
    rows_per_tile = 128 // dim
    n_chunks = n // _CHUNK
    chunks_per_worker = n_chunks // _NUM_WORKERS
    chunks_per_col = batch // _CHUNK
    assert n % (_NUM_WORKERS * _CHUNK) == 0 and batch % _CHUNK == 0
    assert chunks_per_worker % 2 == 0

    mesh = plsc.VectorSubcoreMesh(core_axis_name="c", subcore_axis_name="s")
    idx = inputs.T.reshape(n).astype(jnp.int32)
    table4 = lookup_table.reshape(vocab // rows_per_tile, 128)

    cp = pltpu.CompilerParams()
    if "needs_layout_passes" in pltpu.CompilerParams.__dataclass_fields__:
        cp = dataclasses.replace(cp, needs_layout_passes=False)

    @pl.kernel(
        out_type=jax.ShapeDtypeStruct((seq * dim, batch), jnp.float32),
        mesh=mesh,
        compiler_params=cp,
        scratch_types=[
            pltpu.VMEM((chunks_per_worker * _CHUNK,), jnp.int32),
            pltpu.VMEM((_CHUNK,), jnp.int32),
            pltpu.VMEM((_CHUNK,), jnp.int32),
            pltpu.VMEM((_CHUNK, 128), jnp.float32),
            pltpu.VMEM((_CHUNK, 128), jnp.float32),
            pltpu.VMEM((dim, _CHUNK), jnp.float32),
            pltpu.VMEM((dim, _CHUNK), jnp.float32),
            pltpu.SemaphoreType.DMA,
            pltpu.SemaphoreType.DMA,
            pltpu.SemaphoreType.DMA,
            pltpu.SemaphoreType.DMA,
        ],
    )
    def gather_kernel(
        table_hbm,
        idx_hbm,
        out_hbm,
        idx_all,
        r_a,
        r_b,
        rows_a,
        rows_b,
        trans_a,
        trans_b,
        sem_ga,
        sem_gb,
        sem_oa,
        sem_ob,
    ):
        wid = lax.axis_index("s") * _NUM_CORES + lax.axis_index("c")
        first = wid * chunks_per_worker
        iota = lax.iota(jnp.int32, _LANES)
        groups = _CHUNK // _LANES

        base0 = pl.multiple_of(first * _CHUNK, _CHUNK)
        pltpu.sync_copy(
            idx_hbm.at[pl.ds(base0, chunks_per_worker * _CHUNK)], idx_all
        )

        def fetch(lc, r_v, rows_v, sem_g):
            off = lc * _CHUNK
            for g in range(groups):
                v = idx_all[pl.ds(off + g * _LANES, _LANES)]
                r_v[pl.ds(g * _LANES, _LANES)] = lax.shift_right_logical(v, 2)
            return pltpu.async_copy(table_hbm.at[r_v], rows_v, sem_g)

        def select_store(lc, rows_v, trans_v, sem_o):
            off = lc * _CHUNK
            for g in range(groups):
                v = idx_all[pl.ds(off + g * _LANES, _LANES)]
                col = (v & (rows_per_tile - 1)) * dim
                row16 = g * _LANES + iota
                for e in range(dim):
                    val = plsc.load_gather(rows_v, [row16, col])
                    trans_v[e, pl.ds(g * _LANES, _LANES)] = val
                    if e + 1 < dim:
                        col = col + 1
            ch = first + lc
            t = ch // chunks_per_col
            b0 = (ch % chunks_per_col) * _CHUNK
            r0 = pl.multiple_of(t * dim, 8)
            return pltpu.async_copy(
                trans_v, out_hbm.at[pl.ds(r0, dim), pl.ds(b0, _CHUNK)], sem_o
            )

        fetch(0, r_a, rows_a, sem_ga)

        @pl.loop(0, chunks_per_worker // 2)
        def _(k):
            lca = 2 * k
            lcb = lca + 1
            fetch(lcb, r_b, rows_b, sem_gb)
            pltpu.make_async_copy(table_hbm.at[pl.ds(0, _CHUNK)], rows_a, sem_ga).wait()

            @pl.when(k > 0)
            def _():
                pltpu.make_async_copy(
                    trans_a, out_hbm.at[pl.ds(0, dim), pl.ds(0, _CHUNK)], sem_oa
                ).wait()

            select_store(lca, rows_a, trans_a, sem_oa)

            @pl.when(k + 1 < chunks_per_worker // 2)
            def _():
                fetch(lca + 2, r_a, rows_a, sem_ga)

            pltpu.make_async_copy(table_hbm.at[pl.ds(0, _CHUNK)], rows_b, sem_gb).wait()

            @pl.when(k > 0)
            def _():
                pltpu.make_async_copy(
                    trans_b, out_hbm.at[pl.ds(0, dim), pl.ds(0, _CHUNK)], sem_ob
                ).wait()

            select_store(lcb, rows_b, trans_b, sem_ob)

        pltpu.make_async_copy(
            trans_a, out_hbm.at[pl.ds(0, dim), pl.ds(0, _CHUNK)], sem_oa
        ).wait()
        pltpu.make_async_copy(
            trans_b, out_hbm.at[pl.ds(0, dim), pl.ds(0, _CHUNK)], sem_ob
        ).wait()

    out = gather_kernel(table4, idx)
    return out.reshape(seq, dim, batch).transpose(2, 0, 1)
